# Initial kernel scaffold; baseline (speedup 1.0000x reference)
import jax
import jax.numpy as jnp
from jax.experimental import pallas as pl


def kernel(x_embed, prompt, prompt_key):
    raise NotImplementedError("write your pallas kernel here")


# speedup vs baseline: 1.0629x; 1.0629x over previous
import jax
import jax.numpy as jnp
from jax.experimental import pallas as pl

_POOL = 10
_TOPK = 2
_BB = 8


def _eprompt_body(x_ref, pk_ref, p_ref,
                  sim_ref, idx_ref, bkn_ref, pkn_ref, xn_ref, rs_ref, bp_ref):
    xm = jnp.max(x_ref[...], axis=1)
    xss = jnp.sum(xm * xm, axis=-1, keepdims=True)
    xn = xm * jax.lax.rsqrt(jnp.maximum(xss, 1e-12))
    pk = pk_ref[...]
    pss = jnp.sum(pk * pk, axis=-1, keepdims=True)
    pkn = pk * jax.lax.rsqrt(jnp.maximum(pss, 1e-12))
    pkn_ref[...] = pkn
    xn_ref[...] = xn
    sim = jax.lax.dot_general(xn, pkn, (((1,), (1,)), ((), ())),
                              preferred_element_type=jnp.float32)
    sim_ref[...] = sim
    cols = jax.lax.broadcasted_iota(jnp.int32, sim.shape, 1)
    v1 = jnp.max(sim, axis=1, keepdims=True)
    i1 = jnp.min(jnp.where(sim == v1, cols, _POOL), axis=1, keepdims=True)
    sim_m = jnp.where(cols == i1, -jnp.inf, sim)
    v2 = jnp.max(sim_m, axis=1, keepdims=True)
    i2 = jnp.min(jnp.where(sim_m == v2, cols, _POOL), axis=1, keepdims=True)
    idx_ref[...] = jnp.concatenate([i1, i2], axis=1)

    p_all = p_ref[...]
    for k, ik in enumerate((i1, i2)):
        gk = jnp.zeros((_BB, pkn.shape[1]), jnp.float32)
        gp = jnp.zeros((_BB, pkn.shape[1]), jnp.float32)
        for p in range(_POOL):
            m = ik == p
            gk = gk + jnp.where(m, pkn[p:p + 1, :], 0.0)
            gp = gp + jnp.where(m, p_all[p:p + 1, :], 0.0)
        bkn_ref[:, k, :] = gk
        bp_ref[:, k, :] = gp

    @pl.when(pl.program_id(0) == 0)
    def _():
        rs_ref[...] = jnp.zeros_like(rs_ref)

    rs_ref[...] = rs_ref[...] + (jnp.sum(v1) + jnp.sum(v2))


def kernel(x_embed, prompt, prompt_key):
    B, L, D = x_embed.shape
    grid = (B // _BB,)
    outs = pl.pallas_call(
        _eprompt_body,
        grid=grid,
        in_specs=[
            pl.BlockSpec((_BB, L, D), lambda i: (i, 0, 0)),
            pl.BlockSpec((_POOL, D), lambda i: (0, 0)),
            pl.BlockSpec((_POOL, D), lambda i: (0, 0)),
        ],
        out_specs=[
            pl.BlockSpec((_BB, _POOL), lambda i: (i, 0)),
            pl.BlockSpec((_BB, _TOPK), lambda i: (i, 0)),
            pl.BlockSpec((_BB, _TOPK, D), lambda i: (i, 0, 0)),
            pl.BlockSpec((_POOL, D), lambda i: (0, 0)),
            pl.BlockSpec((_BB, D), lambda i: (i, 0)),
            pl.BlockSpec((1, 1), lambda i: (0, 0)),
            pl.BlockSpec((_BB, _TOPK, D), lambda i: (i, 0, 0)),
        ],
        out_shape=[
            jax.ShapeDtypeStruct((B, _POOL), jnp.float32),
            jax.ShapeDtypeStruct((B, _TOPK), jnp.int32),
            jax.ShapeDtypeStruct((B, _TOPK, D), jnp.float32),
            jax.ShapeDtypeStruct((_POOL, D), jnp.float32),
            jax.ShapeDtypeStruct((B, D), jnp.float32),
            jax.ShapeDtypeStruct((1, 1), jnp.float32),
            jax.ShapeDtypeStruct((B, _TOPK, D), jnp.float32),
        ],
    )(x_embed, prompt_key, prompt)
    sim, idx, bkn, pkn, xn, rs, bp = outs
    reduce_sim = (rs[0, 0] / B).astype(jnp.float32).reshape(())
    return (sim, idx, bkn, pkn, xn, reduce_sim, bp)
